# Initial kernel scaffold; baseline (speedup 1.0000x reference)
import jax
import jax.numpy as jnp
from jax.experimental import pallas as pl


def kernel(decoder_states, encoder_states, step):
    raise NotImplementedError("write your pallas kernel here")


# speedup vs baseline: 2.4848x; 2.4848x over previous
import functools

import jax
import jax.numpy as jnp
from jax import lax
from jax.experimental import pallas as pl
from jax.experimental.pallas import tpu as pltpu
from jax.experimental.pallas import tpu_sc as plsc

NC, NS, L = 2, 16, 16
NW = NC * NS


def kernel(decoder_states, encoder_states, step):
    batch, enc, _ = encoder_states.shape
    _, dec, _ = decoder_states.shape

    value = (jnp.asarray(step, jnp.float32) + 2.0) * jnp.ones((L,), jnp.float32)

    rows_total = batch * dec
    rpw = rows_total // NW
    chunk = L
    blk = chunk * enc
    n_chunks = rpw // chunk
    reuses = n_chunks // 2
    ext = blk + 32 * reuses
    wpb = dec // rpw

    mesh = plsc.VectorSubcoreMesh(core_axis_name="c", subcore_axis_name="s")

    @functools.partial(
        pl.kernel,
        mesh=mesh,
        out_type=jax.ShapeDtypeStruct((rows_total * enc,), jnp.float32),
        scratch_types=[
            pltpu.VMEM((L,), jnp.float32),
            pltpu.VMEM((ext,), jnp.float32),
            pltpu.VMEM((ext,), jnp.float32),
            pltpu.SemaphoreType.DMA,
            pltpu.SemaphoreType.DMA,
        ],
    )
    def sc_fill(val_hbm, out_hbm, val_v, buf0, buf1, sem0, sem1):
        wid = lax.axis_index("s") * NC + lax.axis_index("c")
        row_base = wid * rpw
        drow_base = row_base % dec

        pltpu.sync_copy(val_hbm, val_v)
        val_vec = val_v[...]
        ninf = jnp.full((L,), -jnp.inf, jnp.float32)
        iota = lax.broadcasted_iota(jnp.int32, (L,), 0)

        his = tuple(16 * s + 32 * (reuses - 1) for s in range(2))

        def init(buf, hi):
            def body(j, carry):
                buf[pl.ds(j * L, L)] = ninf
                return carry

            lax.fori_loop(0, ext // L, body, 0)
            anchor = drow_base + hi
            for i in range(chunk):
                buf[pl.ds(anchor + i * enc, L)] = jnp.where(
                    iota == i, val_vec, ninf
                )

        init(buf0, his[0])
        init(buf1, his[1])

        bufs = (buf0, buf1)
        sems = (sem0, sem1)
        copies = [None, None]
        for c in range(n_chunks):
            s = c % 2
            if copies[s] is not None:
                copies[s].wait()
            src_start = his[s] - 16 * c
            copies[s] = pltpu.async_copy(
                bufs[s].at[pl.ds(src_start, blk)],
                out_hbm.at[pl.ds((row_base + c * chunk) * enc, blk)],
                sems[s],
            )
        for s in range(2):
            if copies[s] is not None:
                copies[s].wait()

    return sc_fill(value).reshape(batch, dec, enc)


# speedup vs baseline: 5.7791x; 2.3258x over previous
import functools

import jax
import jax.numpy as jnp
from jax import lax
from jax.experimental import pallas as pl
from jax.experimental.pallas import tpu as pltpu
from jax.experimental.pallas import tpu_sc as plsc

NC, NS, L = 2, 16, 16
NW = NC * NS
TILE = 128


def kernel(decoder_states, encoder_states, step):
    batch, enc, _ = encoder_states.shape
    _, dec, _ = decoder_states.shape

    value = (jnp.asarray(step, jnp.float32) + 2.0) * jnp.ones((L,), jnp.float32)

    rows_total = batch * dec
    rpw = rows_total // NW
    chunk = L
    n_chunks = rpw // chunk
    n_pat = TILE // L

    mesh = plsc.VectorSubcoreMesh(core_axis_name="c", subcore_axis_name="s")

    @functools.partial(
        pl.kernel,
        mesh=mesh,
        out_type=jax.ShapeDtypeStruct((batch, dec, enc), jnp.float32),
        scratch_types=[
            pltpu.VMEM((L,), jnp.float32),
            pltpu.VMEM((chunk, enc), jnp.float32),
            pltpu.VMEM((L, n_pat * TILE), jnp.float32),
            pltpu.SemaphoreType.DMA,
            pltpu.SemaphoreType.DMA,
            pltpu.SemaphoreType.DMA,
        ],
    )
    def sc_fill(val_hbm, out_hbm, val_v, blk_v, pat_v, sem0, sem1, psem):
        wid = lax.axis_index("s") * NC + lax.axis_index("c")
        row_base = wid * rpw
        b = row_base // dec
        drow_base = row_base % dec
        drow_base = pl.multiple_of(drow_base, TILE)

        pltpu.sync_copy(val_hbm, val_v)
        val_vec = val_v[...]
        ninf = jnp.full((L,), -jnp.inf, jnp.float32)
        iota = lax.broadcasted_iota(jnp.int32, (L,), 0)

        def init_blk(r, carry):
            def body(j, carry2):
                blk_v[r, pl.ds(j * L, L)] = ninf
                return carry2

            return lax.fori_loop(0, enc // L, body, carry)

        lax.fori_loop(0, chunk, init_blk, 0)
        def init_pat(r, carry):
            def body(j, carry2):
                pat_v[r, pl.ds(j * L, L)] = ninf
                return carry2

            return lax.fori_loop(0, n_pat * TILE // L, body, carry)

        lax.fori_loop(0, chunk, init_pat, 0)
        for p in range(n_pat):
            for i in range(L):
                pat_v[i, pl.ds(p * (TILE + L), L)] = jnp.where(
                    iota == i, val_vec, ninf
                )

        def patch_dst(c):
            r0 = drow_base + c * chunk
            ct = drow_base + (c // n_pat) * TILE
            return out_hbm.at[b, pl.ds(r0, L), pl.ds(pl.multiple_of(ct, TILE), TILE)]

        sems = (sem0, sem1)
        blk_copies = [None, None]
        patch_copies = []
        for c in range(n_chunks):
            s = c % 2
            if blk_copies[s] is not None:
                blk_copies[s].wait()
                patch_copies.append(
                    pltpu.async_copy(
                        pat_v.at[:, pl.ds(((c - 2) % n_pat) * TILE, TILE)],
                        patch_dst(c - 2),
                        psem,
                    )
                )
            r0 = drow_base + c * chunk
            blk_copies[s] = pltpu.async_copy(
                blk_v, out_hbm.at[b, pl.ds(r0, chunk), :], sems[s]
            )
        for c in (n_chunks - 2, n_chunks - 1):
            blk_copies[c % 2].wait()
            patch_copies.append(
                pltpu.async_copy(
                    pat_v.at[:, pl.ds((c % n_pat) * TILE, TILE)],
                    patch_dst(c),
                    psem,
                )
            )
        for h in patch_copies:
            h.wait()

    return sc_fill(value)
